# baseline (speedup 1.0000x reference)
import functools

import jax
import jax.numpy as jnp
from jax import lax
from jax.experimental import pallas as pl
from jax.experimental.pallas import tpu as pltpu
from jax.experimental.pallas import tpu_sc as plsc

N = 2000; E = 256000; R = 32; IN = 256; H = 128; OUT = 128; K = 1600

NC, NS, L = 2, 16, 16
NW = NC * NS
EPT = E // NW
CH = 80
NCHUNK = EPT // CH
NB2 = K * R
NB2P = 51456
ZCA = NB2P // NS
KP = 1664
KPS = KP // NS
KOC = 160
NIDP = 2048

_HI = jax.lax.Precision.HIGHEST


def _sc_mesh():
    return plsc.VectorSubcoreMesh(
        core_axis_name="c", subcore_axis_name="s",
        num_cores=NC, num_subcores=NS)


_SC_PARAMS = pltpu.CompilerParams(needs_layout_passes=False)


@functools.partial(
    pl.kernel,
    out_type=(
        jax.ShapeDtypeStruct((E,), jnp.int32),
        jax.ShapeDtypeStruct((E,), jnp.int32),
        jax.ShapeDtypeStruct((NC * NB2P,), jnp.float32),
    ),
    mesh=_sc_mesh(),
    compiler_params=_SC_PARAMS,
    scratch_types=[
        pltpu.VMEM((NIDP,), jnp.int32),
        pltpu.VMEM((CH,), jnp.int32),
        pltpu.VMEM((CH,), jnp.int32),
        pltpu.VMEM((CH,), jnp.int32),
        pltpu.VMEM((CH,), jnp.int32),
        pltpu.VMEM((CH,), jnp.int32),
        pltpu.VMEM((CH,), jnp.float32),
        pltpu.VMEM((ZCA,), jnp.float32),
        pltpu.VMEM_SHARED((NB2P,), jnp.float32),
    ],
)
def _sc_relabel(src_h, dst_h, et_h, nid_h, comb2_h, gidx2_h, cnt2p_h,
                nid_v, src_v, dst_v, et_v, comb_v, gidx_v, ones_v, stage_v,
                cnt_sh):
    c = lax.axis_index("c")
    s = lax.axis_index("s")
    wid = c * NS + s

    def _fill(i, _):
        stage_v[pl.ds(i * L, L)] = jnp.zeros((L,), jnp.float32)
        return 0
    lax.fori_loop(0, ZCA // L, _fill, 0)
    for j in range(CH // L):
        ones_v[pl.ds(j * L, L)] = jnp.ones((L,), jnp.float32)
    pltpu.sync_copy(stage_v, cnt_sh.at[pl.ds(s * ZCA, ZCA)])
    pltpu.sync_copy(nid_h, nid_v)
    plsc.subcore_barrier()

    base = wid * EPT

    def _chunk(k, _):
        off = base + k * CH
        pltpu.sync_copy(src_h.at[pl.ds(off, CH)], src_v)
        pltpu.sync_copy(dst_h.at[pl.ds(off, CH)], dst_v)
        pltpu.sync_copy(et_h.at[pl.ds(off, CH)], et_v)
        for j in range(CH // L):
            sl = pl.ds(j * L, L)
            ns = plsc.load_gather(nid_v, [src_v[sl]])
            nd = plsc.load_gather(nid_v, [dst_v[sl]])
            e16 = et_v[sl]
            valid = (ns >= 0) & (nd >= 0)
            comb_v[sl] = jnp.where(valid, nd * R + e16, NB2)
            gidx_v[sl] = jnp.where(valid, ns * R + e16, 0)
        pltpu.sync_copy(comb_v, comb2_h.at[pl.ds(off, CH)])
        pltpu.sync_copy(gidx_v, gidx2_h.at[pl.ds(off, CH)])
        pltpu.sync_copy(ones_v, cnt_sh.at[comb_v], add=True)
        return 0
    lax.fori_loop(0, NCHUNK, _chunk, 0)

    plsc.subcore_barrier()
    pltpu.sync_copy(cnt_sh.at[pl.ds(s * ZCA, ZCA)], stage_v)
    pltpu.sync_copy(stage_v, cnt2p_h.at[pl.ds(c * NB2P + s * ZCA, ZCA)])


@functools.partial(
    pl.kernel,
    out_type=jax.ShapeDtypeStruct((NC, K, OUT), jnp.float32),
    mesh=_sc_mesh(),
    compiler_params=_SC_PARAMS,
    scratch_types=[
        pltpu.VMEM((NB2P,), jnp.float32),
        pltpu.VMEM((CH,), jnp.int32),
        pltpu.VMEM((CH,), jnp.int32),
        pltpu.VMEM((CH,), jnp.int32),
        pltpu.VMEM((CH,), jnp.int32),
        pltpu.VMEM((CH,), jnp.int32),
        pltpu.VMEM((CH, OUT), jnp.float32),
        pltpu.VMEM((CH, OUT), jnp.float32),
        pltpu.VMEM((CH, OUT), jnp.float32),
        pltpu.VMEM((KOC, OUT), jnp.float32),
        pltpu.SemaphoreType.DMA,
        pltpu.SemaphoreType.DMA,
        pltpu.VMEM_SHARED((KP, OUT), jnp.float32),
    ],
)
def _sc_messages(xw2_h, gidx2_h, comb2_h, recip_h, aggp_h,
                 recip_v, gidx_v0, gidx_v1, comb_v0, comb_v1, dstr_v,
                 rows_v0, rows_v1, srows_v, stage_v, gsem0, gsem1, agg_sh):
    c = lax.axis_index("c")
    s = lax.axis_index("s")
    wid = c * NS + s

    def _fill(i, _):
        stage_v[i // (OUT // L),
                pl.ds((i % (OUT // L)) * L, L)] = jnp.zeros((L,), jnp.float32)
        return 0
    lax.fori_loop(0, KOC * (OUT // L), _fill, 0)
    pltpu.sync_copy(stage_v.at[pl.ds(0, KPS)], agg_sh.at[pl.ds(s * KPS, KPS)])
    pltpu.sync_copy(recip_h, recip_v)
    plsc.subcore_barrier()

    base = wid * EPT
    gidx_b = (gidx_v0, gidx_v1)
    comb_b = (comb_v0, comb_v1)
    rows_b = (rows_v0, rows_v1)
    gsem_b = (gsem0, gsem1)

    pltpu.sync_copy(gidx2_h.at[pl.ds(base, CH)], gidx_v0)
    pltpu.sync_copy(comb2_h.at[pl.ds(base, CH)], comb_v0)
    pltpu.async_copy(xw2_h.at[gidx_v0], rows_v0, gsem0)

    def _pair(kk, _):
        for b in range(2):
            k = 2 * kk + b
            nxt = (k + 1) * CH
            nxt = jnp.where(nxt >= EPT, 0, nxt)
            pltpu.sync_copy(gidx2_h.at[pl.ds(base + nxt, CH)],
                            gidx_b[1 - b])
            pltpu.sync_copy(comb2_h.at[pl.ds(base + nxt, CH)],
                            comb_b[1 - b])
            pltpu.make_async_copy(xw2_h.at[gidx_b[b]], rows_b[b],
                                  gsem_b[b]).wait()
            pltpu.async_copy(xw2_h.at[gidx_b[1 - b]], rows_b[1 - b],
                             gsem_b[1 - b])
            for j in range(CH // L):
                sl = pl.ds(j * L, L)
                c16 = comb_b[b][sl]
                dstr_v[sl] = lax.shift_right_logical(c16, 5)
                sc16 = plsc.load_gather(recip_v, [c16])
                ridx = lax.iota(jnp.int32, L) + (j * L)
                for d in range(OUT):
                    cidx = jnp.full((L,), d, jnp.int32)
                    v = plsc.load_gather(rows_b[b], [ridx, cidx])
                    plsc.store_scatter(srows_v, [ridx, cidx], v * sc16)
            pltpu.sync_copy(srows_v, agg_sh.at[dstr_v], add=True)
        return 0
    lax.fori_loop(0, NCHUNK // 2, _pair, 0)
    pltpu.make_async_copy(xw2_h.at[gidx_v0], rows_v0, gsem0).wait()

    plsc.subcore_barrier()

    @pl.when(s < K // KOC)
    def _():
        ksl = pl.ds(s * KOC, KOC)
        pltpu.sync_copy(agg_sh.at[ksl], stage_v)
        pltpu.sync_copy(stage_v, aggp_h.at[c, ksl])


def _mm_body(a_ref, b_ref, o_ref):
    o_ref[...] = jax.lax.dot_general(
        a_ref[...], b_ref[...], (((1,), (0,)), ((), ())),
        precision=_HI, preferred_element_type=jnp.float32)


def _tc_xw2(hp, wall2):
    return pl.pallas_call(
        _mm_body,
        grid=(8,),
        in_specs=[pl.BlockSpec((K, H), lambda i: (0, 0)),
                  pl.BlockSpec((H, 512), lambda i: (0, i))],
        out_specs=pl.BlockSpec((K, 512), lambda i: (0, i)),
        out_shape=jax.ShapeDtypeStruct((K, R * OUT), jnp.float32),
    )(hp, wall2)


def _root_body(a_ref, b_ref, bias_ref, o_ref):
    o_ref[...] = jax.lax.dot_general(
        a_ref[...], b_ref[...], (((1,), (0,)), ((), ())),
        precision=_HI, preferred_element_type=jnp.float32) + bias_ref[...]


def _tc_root(hp, root, bias):
    return pl.pallas_call(
        _root_body,
        in_specs=[pl.BlockSpec((K, H), lambda: (0, 0)),
                  pl.BlockSpec((H, OUT), lambda: (0, 0)),
                  pl.BlockSpec((1, OUT), lambda: (0, 0))],
        out_specs=pl.BlockSpec((K, OUT), lambda: (0, 0)),
        out_shape=jax.ShapeDtypeStruct((K, OUT), jnp.float32),
    )(hp, root, bias.reshape(1, OUT))


def _recip_body(c_ref, o_ref):
    cnt = c_ref[0] + c_ref[1]
    i0 = lax.broadcasted_iota(jnp.int32, (NB2P // 128, 128), 0)
    i1 = lax.broadcasted_iota(jnp.int32, (NB2P // 128, 128), 1)
    flat = i0 * 128 + i1
    r = 1.0 / jnp.maximum(cnt, 1.0)
    o_ref[...] = jnp.where(flat < NB2, r, 0.0)


def _tc_recip(cnt2p):
    return pl.pallas_call(
        _recip_body,
        in_specs=[pl.BlockSpec((NC, NB2P // 128, 128), lambda: (0, 0, 0))],
        out_specs=pl.BlockSpec((NB2P // 128, 128), lambda: (0, 0)),
        out_shape=jax.ShapeDtypeStruct((NB2P // 128, 128), jnp.float32),
    )(cnt2p.reshape(NC, NB2P // 128, 128)).reshape(NB2P)


_FC = 4096
_NF = (K * OUT) // _FC


def _fin_body(w_ref, p0_ref, p1_ref, hr_ref, b_ref, o_ref):
    i = pl.program_id(0)
    f = p0_ref[...] + p1_ref[...] + hr_ref[...]
    part = jax.lax.dot_general(
        w_ref[...], f, (((1,), (0,)), ((), ())),
        precision=_HI, preferred_element_type=jnp.float32)

    @pl.when(i == 0)
    def _():
        o_ref[...] = jnp.zeros_like(o_ref)

    o_ref[...] += part

    @pl.when(i == _NF - 1)
    def _():
        o_ref[...] += b_ref[...]


def _tc_final(lin_W, p0, p1, hrootflat, lin_b):
    return pl.pallas_call(
        _fin_body,
        grid=(_NF,),
        in_specs=[pl.BlockSpec((128, _FC), lambda i: (0, i)),
                  pl.BlockSpec((_FC, 1), lambda i: (i, 0)),
                  pl.BlockSpec((_FC, 1), lambda i: (i, 0)),
                  pl.BlockSpec((_FC, 1), lambda i: (i, 0)),
                  pl.BlockSpec((128, 1), lambda i: (0, 0))],
        out_specs=pl.BlockSpec((128, 1), lambda i: (0, 0)),
        out_shape=jax.ShapeDtypeStruct((128, 1), jnp.float32),
    )(lin_W, p0, p1, hrootflat, lin_b.reshape(128, 1))


def kernel(x, edge_index, edge_type, conv1_W, conv1_root, conv1_bias,
           gnn_rel_W, gnn_rel_b, gnn_root_W, conv2_W, conv2_root, conv2_bias,
           lin_W, lin_b):
    src, dst = edge_index[0], edge_index[1]

    valid0 = jnp.ones((E,), dtype=bool)
    xw = jnp.einsum('nc,rcd->nrd', x, conv1_W)
    m = xw[src, edge_type]
    comb = jnp.where(valid0, dst * R + edge_type, 0)
    ones = valid0.astype(x.dtype)
    cnt = jax.ops.segment_sum(ones, comb, num_segments=N * R)
    denom = jnp.maximum(cnt[comb], 1.0)
    m = m * (ones / denom)[:, None]
    agg = jax.ops.segment_sum(m, jnp.where(valid0, dst, 0), num_segments=N)
    h = agg + x @ conv1_root + conv1_bias

    agg2 = jax.ops.segment_sum(h[src], dst, num_segments=N)
    score = (agg2 @ gnn_rel_W + gnn_rel_b + h @ gnn_root_W).reshape(-1)
    score = jnp.tanh(score)
    topv, perm = jax.lax.top_k(score, K)
    hp = h[perm] * topv[:, None]
    new_id = jnp.full((N,), -1, jnp.int32).at[perm].set(
        jnp.arange(K, dtype=jnp.int32))

    nid_pad = jnp.full((NIDP,), -1, jnp.int32).at[:N].set(new_id)
    comb2, gidx2, cnt2p = _sc_relabel(src, dst, edge_type, nid_pad)
    recip2 = _tc_recip(cnt2p.reshape(NC, NB2P))

    wall2 = jnp.transpose(conv2_W, (1, 0, 2)).reshape(H, R * OUT)
    xw2 = _tc_xw2(hp, wall2).reshape(K * R, OUT)
    hproot = _tc_root(hp, conv2_root, conv2_bias)

    aggp = _sc_messages(xw2, gidx2, comb2, recip2)

    p0 = aggp[0].reshape(K * OUT, 1)
    p1 = aggp[1].reshape(K * OUT, 1)
    hrf = hproot.reshape(K * OUT, 1)
    out = _tc_final(lin_W, p0, p1, hrf, lin_b)
    return out.reshape(128)


# speedup vs baseline: 1.0014x; 1.0014x over previous
import functools

import jax
import jax.numpy as jnp
from jax import lax
from jax.experimental import pallas as pl
from jax.experimental.pallas import tpu as pltpu
from jax.experimental.pallas import tpu_sc as plsc

N = 2000; E = 256000; R = 32; IN = 256; H = 128; OUT = 128; K = 1600

NC, NS, L = 2, 16, 16
NW = NC * NS
EPT = E // NW
CH = 80
NCHUNK = EPT // CH
NB2 = K * R
NB2P = 51456
ZCA = NB2P // NS
KP = 1664
KPS = KP // NS
KOC = 160
NIDP = 2048

_HI = jax.lax.Precision.HIGHEST


def _sc_mesh():
    return plsc.VectorSubcoreMesh(
        core_axis_name="c", subcore_axis_name="s",
        num_cores=NC, num_subcores=NS)


_SC_PARAMS = pltpu.CompilerParams(needs_layout_passes=False)


@functools.partial(
    pl.kernel,
    out_type=(
        jax.ShapeDtypeStruct((E,), jnp.int32),
        jax.ShapeDtypeStruct((E,), jnp.int32),
        jax.ShapeDtypeStruct((NC * NB2P,), jnp.float32),
    ),
    mesh=_sc_mesh(),
    compiler_params=_SC_PARAMS,
    scratch_types=[
        pltpu.VMEM((NIDP,), jnp.int32),
        pltpu.VMEM((CH,), jnp.int32),
        pltpu.VMEM((CH,), jnp.int32),
        pltpu.VMEM((CH,), jnp.int32),
        pltpu.VMEM((CH,), jnp.int32),
        pltpu.VMEM((CH,), jnp.int32),
        pltpu.VMEM((CH,), jnp.float32),
        pltpu.VMEM((ZCA,), jnp.float32),
        pltpu.VMEM_SHARED((NB2P,), jnp.float32),
    ],
)
def _sc_relabel(src_h, dst_h, et_h, nid_h, comb2_h, gidx2_h, cnt2p_h,
                nid_v, src_v, dst_v, et_v, comb_v, gidx_v, ones_v, stage_v,
                cnt_sh):
    c = lax.axis_index("c")
    s = lax.axis_index("s")
    wid = c * NS + s

    def _fill(i, _):
        stage_v[pl.ds(i * L, L)] = jnp.zeros((L,), jnp.float32)
        return 0
    lax.fori_loop(0, ZCA // L, _fill, 0)
    for j in range(CH // L):
        ones_v[pl.ds(j * L, L)] = jnp.ones((L,), jnp.float32)
    pltpu.sync_copy(stage_v, cnt_sh.at[pl.ds(s * ZCA, ZCA)])
    pltpu.sync_copy(nid_h, nid_v)
    plsc.subcore_barrier()

    base = wid * EPT

    def _chunk(k, _):
        off = base + k * CH
        pltpu.sync_copy(src_h.at[pl.ds(off, CH)], src_v)
        pltpu.sync_copy(dst_h.at[pl.ds(off, CH)], dst_v)
        pltpu.sync_copy(et_h.at[pl.ds(off, CH)], et_v)
        for j in range(CH // L):
            sl = pl.ds(j * L, L)
            ns = plsc.load_gather(nid_v, [src_v[sl]])
            nd = plsc.load_gather(nid_v, [dst_v[sl]])
            e16 = et_v[sl]
            valid = (ns >= 0) & (nd >= 0)
            comb_v[sl] = jnp.where(valid, nd * R + e16, NB2)
            gidx_v[sl] = jnp.where(valid, ns * R + e16, 0)
        pltpu.sync_copy(comb_v, comb2_h.at[pl.ds(off, CH)])
        pltpu.sync_copy(gidx_v, gidx2_h.at[pl.ds(off, CH)])
        pltpu.sync_copy(ones_v, cnt_sh.at[comb_v], add=True)
        return 0
    lax.fori_loop(0, NCHUNK, _chunk, 0)

    plsc.subcore_barrier()
    pltpu.sync_copy(cnt_sh.at[pl.ds(s * ZCA, ZCA)], stage_v)
    pltpu.sync_copy(stage_v, cnt2p_h.at[pl.ds(c * NB2P + s * ZCA, ZCA)])


@functools.partial(
    pl.kernel,
    out_type=jax.ShapeDtypeStruct((NC, K, OUT), jnp.float32),
    mesh=_sc_mesh(),
    compiler_params=_SC_PARAMS,
    scratch_types=[
        pltpu.VMEM((NB2P,), jnp.float32),
        pltpu.VMEM((CH,), jnp.int32),
        pltpu.VMEM((CH,), jnp.int32),
        pltpu.VMEM((CH,), jnp.int32),
        pltpu.VMEM((CH,), jnp.int32),
        pltpu.VMEM((CH,), jnp.int32),
        pltpu.VMEM((CH, OUT), jnp.float32),
        pltpu.VMEM((CH, OUT), jnp.float32),
        pltpu.VMEM((KOC, OUT), jnp.float32),
        pltpu.SemaphoreType.DMA,
        pltpu.SemaphoreType.DMA,
        pltpu.VMEM_SHARED((KP, OUT), jnp.float32),
    ],
)
def _sc_messages(xw2_h, gidx2_h, comb2_h, recip_h, aggp_h,
                 recip_v, gidx_v0, gidx_v1, comb_v0, comb_v1, dstr_v,
                 rows_v0, rows_v1, stage_v, gsem0, gsem1, agg_sh):
    c = lax.axis_index("c")
    s = lax.axis_index("s")
    wid = c * NS + s

    def _fill(i, _):
        stage_v[i // (OUT // L),
                pl.ds((i % (OUT // L)) * L, L)] = jnp.zeros((L,), jnp.float32)
        return 0
    lax.fori_loop(0, KOC * (OUT // L), _fill, 0)
    pltpu.sync_copy(stage_v.at[pl.ds(0, KPS)], agg_sh.at[pl.ds(s * KPS, KPS)])
    pltpu.sync_copy(recip_h, recip_v)
    plsc.subcore_barrier()

    base = wid * EPT
    gidx_b = (gidx_v0, gidx_v1)
    comb_b = (comb_v0, comb_v1)
    rows_b = (rows_v0, rows_v1)
    gsem_b = (gsem0, gsem1)

    pltpu.sync_copy(gidx2_h.at[pl.ds(base, CH)], gidx_v0)
    pltpu.sync_copy(comb2_h.at[pl.ds(base, CH)], comb_v0)
    pltpu.async_copy(xw2_h.at[gidx_v0], rows_v0, gsem0)

    def _pair(kk, _):
        for b in range(2):
            k = 2 * kk + b
            nxt = (k + 1) * CH
            nxt = jnp.where(nxt >= EPT, 0, nxt)
            pltpu.sync_copy(gidx2_h.at[pl.ds(base + nxt, CH)],
                            gidx_b[1 - b])
            pltpu.sync_copy(comb2_h.at[pl.ds(base + nxt, CH)],
                            comb_b[1 - b])
            pltpu.make_async_copy(xw2_h.at[gidx_b[b]], rows_b[b],
                                  gsem_b[b]).wait()
            pltpu.async_copy(xw2_h.at[gidx_b[1 - b]], rows_b[1 - b],
                             gsem_b[1 - b])
            for j in range(CH // L):
                sl = pl.ds(j * L, L)
                c16 = comb_b[b][sl]
                dstr_v[sl] = lax.shift_right_logical(c16, 5)
                sc16 = plsc.load_gather(recip_v, [c16])
                for r in range(L):
                    spl = jnp.full((L,), sc16[r], jnp.float32)
                    i = j * L + r
                    for d in range(OUT // L):
                        dl = pl.ds(d * L, L)
                        rows_b[b][i, dl] = rows_b[b][i, dl] * spl
            pltpu.sync_copy(rows_b[b], agg_sh.at[dstr_v], add=True)
        return 0
    lax.fori_loop(0, NCHUNK // 2, _pair, 0)
    pltpu.make_async_copy(xw2_h.at[gidx_v0], rows_v0, gsem0).wait()

    plsc.subcore_barrier()

    @pl.when(s < K // KOC)
    def _():
        ksl = pl.ds(s * KOC, KOC)
        pltpu.sync_copy(agg_sh.at[ksl], stage_v)
        pltpu.sync_copy(stage_v, aggp_h.at[c, ksl])


def _mm_body(a_ref, b_ref, o_ref):
    o_ref[...] = jax.lax.dot_general(
        a_ref[...], b_ref[...], (((1,), (0,)), ((), ())),
        precision=_HI, preferred_element_type=jnp.float32)


def _tc_xw2(hp, wall2):
    return pl.pallas_call(
        _mm_body,
        grid=(8,),
        in_specs=[pl.BlockSpec((K, H), lambda i: (0, 0)),
                  pl.BlockSpec((H, 512), lambda i: (0, i))],
        out_specs=pl.BlockSpec((K, 512), lambda i: (0, i)),
        out_shape=jax.ShapeDtypeStruct((K, R * OUT), jnp.float32),
    )(hp, wall2)


def _root_body(a_ref, b_ref, bias_ref, o_ref):
    o_ref[...] = jax.lax.dot_general(
        a_ref[...], b_ref[...], (((1,), (0,)), ((), ())),
        precision=_HI, preferred_element_type=jnp.float32) + bias_ref[...]


def _tc_root(hp, root, bias):
    return pl.pallas_call(
        _root_body,
        in_specs=[pl.BlockSpec((K, H), lambda: (0, 0)),
                  pl.BlockSpec((H, OUT), lambda: (0, 0)),
                  pl.BlockSpec((1, OUT), lambda: (0, 0))],
        out_specs=pl.BlockSpec((K, OUT), lambda: (0, 0)),
        out_shape=jax.ShapeDtypeStruct((K, OUT), jnp.float32),
    )(hp, root, bias.reshape(1, OUT))


def _recip_body(c_ref, o_ref):
    cnt = c_ref[0] + c_ref[1]
    i0 = lax.broadcasted_iota(jnp.int32, (NB2P // 128, 128), 0)
    i1 = lax.broadcasted_iota(jnp.int32, (NB2P // 128, 128), 1)
    flat = i0 * 128 + i1
    r = 1.0 / jnp.maximum(cnt, 1.0)
    o_ref[...] = jnp.where(flat < NB2, r, 0.0)


def _tc_recip(cnt2p):
    return pl.pallas_call(
        _recip_body,
        in_specs=[pl.BlockSpec((NC, NB2P // 128, 128), lambda: (0, 0, 0))],
        out_specs=pl.BlockSpec((NB2P // 128, 128), lambda: (0, 0)),
        out_shape=jax.ShapeDtypeStruct((NB2P // 128, 128), jnp.float32),
    )(cnt2p.reshape(NC, NB2P // 128, 128)).reshape(NB2P)


_FC = 4096
_NF = (K * OUT) // _FC


def _fin_body(w_ref, p0_ref, p1_ref, hr_ref, b_ref, o_ref):
    i = pl.program_id(0)
    f = p0_ref[...] + p1_ref[...] + hr_ref[...]
    part = jax.lax.dot_general(
        w_ref[...], f, (((1,), (0,)), ((), ())),
        precision=_HI, preferred_element_type=jnp.float32)

    @pl.when(i == 0)
    def _():
        o_ref[...] = jnp.zeros_like(o_ref)

    o_ref[...] += part

    @pl.when(i == _NF - 1)
    def _():
        o_ref[...] += b_ref[...]


def _tc_final(lin_W, p0, p1, hrootflat, lin_b):
    return pl.pallas_call(
        _fin_body,
        grid=(_NF,),
        in_specs=[pl.BlockSpec((128, _FC), lambda i: (0, i)),
                  pl.BlockSpec((_FC, 1), lambda i: (i, 0)),
                  pl.BlockSpec((_FC, 1), lambda i: (i, 0)),
                  pl.BlockSpec((_FC, 1), lambda i: (i, 0)),
                  pl.BlockSpec((128, 1), lambda i: (0, 0))],
        out_specs=pl.BlockSpec((128, 1), lambda i: (0, 0)),
        out_shape=jax.ShapeDtypeStruct((128, 1), jnp.float32),
    )(lin_W, p0, p1, hrootflat, lin_b.reshape(128, 1))


def kernel(x, edge_index, edge_type, conv1_W, conv1_root, conv1_bias,
           gnn_rel_W, gnn_rel_b, gnn_root_W, conv2_W, conv2_root, conv2_bias,
           lin_W, lin_b):
    src, dst = edge_index[0], edge_index[1]

    valid0 = jnp.ones((E,), dtype=bool)
    xw = jnp.einsum('nc,rcd->nrd', x, conv1_W)
    m = xw[src, edge_type]
    comb = jnp.where(valid0, dst * R + edge_type, 0)
    ones = valid0.astype(x.dtype)
    cnt = jax.ops.segment_sum(ones, comb, num_segments=N * R)
    denom = jnp.maximum(cnt[comb], 1.0)
    m = m * (ones / denom)[:, None]
    agg = jax.ops.segment_sum(m, jnp.where(valid0, dst, 0), num_segments=N)
    h = agg + x @ conv1_root + conv1_bias

    agg2 = jax.ops.segment_sum(h[src], dst, num_segments=N)
    score = (agg2 @ gnn_rel_W + gnn_rel_b + h @ gnn_root_W).reshape(-1)
    score = jnp.tanh(score)
    topv, perm = jax.lax.top_k(score, K)
    hp = h[perm] * topv[:, None]
    new_id = jnp.full((N,), -1, jnp.int32).at[perm].set(
        jnp.arange(K, dtype=jnp.int32))

    nid_pad = jnp.full((NIDP,), -1, jnp.int32).at[:N].set(new_id)
    comb2, gidx2, cnt2p = _sc_relabel(src, dst, edge_type, nid_pad)
    recip2 = _tc_recip(cnt2p.reshape(NC, NB2P))

    wall2 = jnp.transpose(conv2_W, (1, 0, 2)).reshape(H, R * OUT)
    xw2 = _tc_xw2(hp, wall2).reshape(K * R, OUT)
    hproot = _tc_root(hp, conv2_root, conv2_bias)

    aggp = _sc_messages(xw2, gidx2, comb2, recip2)

    p0 = aggp[0].reshape(K * OUT, 1)
    p1 = aggp[1].reshape(K * OUT, 1)
    hrf = hproot.reshape(K * OUT, 1)
    out = _tc_final(lin_W, p0, p1, hrf, lin_b)
    return out.reshape(128)


# speedup vs baseline: 1.0016x; 1.0003x over previous
import functools

import jax
import jax.numpy as jnp
from jax import lax
from jax.experimental import pallas as pl
from jax.experimental.pallas import tpu as pltpu
from jax.experimental.pallas import tpu_sc as plsc

N = 2000; E = 256000; R = 32; IN = 256; H = 128; OUT = 128; K = 1600

NC, NS, L = 2, 16, 16
NW = NC * NS
EPT = E // NW
CH = 80
NCHUNK = EPT // CH
NB2 = K * R
NB2P = 51456
ZCA = NB2P // NS
KP = 1664
KPS = KP // NS
KOC = 160
NIDP = 2048

_HI = jax.lax.Precision.HIGHEST


def _sc_mesh():
    return plsc.VectorSubcoreMesh(
        core_axis_name="c", subcore_axis_name="s",
        num_cores=NC, num_subcores=NS)


_SC_PARAMS = pltpu.CompilerParams(needs_layout_passes=False)


@functools.partial(
    pl.kernel,
    out_type=(
        jax.ShapeDtypeStruct((E,), jnp.int32),
        jax.ShapeDtypeStruct((E,), jnp.int32),
        jax.ShapeDtypeStruct((NC * NB2P,), jnp.float32),
    ),
    mesh=_sc_mesh(),
    compiler_params=_SC_PARAMS,
    scratch_types=[
        pltpu.VMEM((NIDP,), jnp.int32),
        pltpu.VMEM((CH,), jnp.int32),
        pltpu.VMEM((CH,), jnp.int32),
        pltpu.VMEM((CH,), jnp.int32),
        pltpu.VMEM((CH,), jnp.int32),
        pltpu.VMEM((CH,), jnp.int32),
        pltpu.VMEM((CH,), jnp.float32),
        pltpu.VMEM((ZCA,), jnp.float32),
        pltpu.VMEM_SHARED((NB2P,), jnp.float32),
    ],
)
def _sc_relabel(src_h, dst_h, et_h, nid_h, comb2_h, gidx2_h, cnt2p_h,
                nid_v, src_v, dst_v, et_v, comb_v, gidx_v, ones_v, stage_v,
                cnt_sh):
    c = lax.axis_index("c")
    s = lax.axis_index("s")
    wid = c * NS + s

    def _fill(i, _):
        stage_v[pl.ds(i * L, L)] = jnp.zeros((L,), jnp.float32)
        return 0
    lax.fori_loop(0, ZCA // L, _fill, 0)
    for j in range(CH // L):
        ones_v[pl.ds(j * L, L)] = jnp.ones((L,), jnp.float32)
    pltpu.sync_copy(stage_v, cnt_sh.at[pl.ds(s * ZCA, ZCA)])
    pltpu.sync_copy(nid_h, nid_v)
    plsc.subcore_barrier()

    base = wid * EPT

    def _chunk(k, _):
        off = base + k * CH
        pltpu.sync_copy(src_h.at[pl.ds(off, CH)], src_v)
        pltpu.sync_copy(dst_h.at[pl.ds(off, CH)], dst_v)
        pltpu.sync_copy(et_h.at[pl.ds(off, CH)], et_v)
        for j in range(CH // L):
            sl = pl.ds(j * L, L)
            ns = plsc.load_gather(nid_v, [src_v[sl]])
            nd = plsc.load_gather(nid_v, [dst_v[sl]])
            e16 = et_v[sl]
            valid = (ns >= 0) & (nd >= 0)
            comb_v[sl] = jnp.where(valid, nd * R + e16, NB2)
            gidx_v[sl] = jnp.where(valid, ns * R + e16, 0)
        pltpu.sync_copy(comb_v, comb2_h.at[pl.ds(off, CH)])
        pltpu.sync_copy(gidx_v, gidx2_h.at[pl.ds(off, CH)])
        pltpu.sync_copy(ones_v, cnt_sh.at[comb_v], add=True)
        return 0
    lax.fori_loop(0, NCHUNK, _chunk, 0)

    plsc.subcore_barrier()
    pltpu.sync_copy(cnt_sh.at[pl.ds(s * ZCA, ZCA)], stage_v)
    pltpu.sync_copy(stage_v, cnt2p_h.at[pl.ds(c * NB2P + s * ZCA, ZCA)])


@functools.partial(
    pl.kernel,
    out_type=jax.ShapeDtypeStruct((NC, K, OUT), jnp.float32),
    mesh=_sc_mesh(),
    compiler_params=_SC_PARAMS,
    scratch_types=[
        pltpu.VMEM((NB2P,), jnp.float32),
        pltpu.VMEM((CH,), jnp.int32),
        pltpu.VMEM((CH,), jnp.int32),
        pltpu.VMEM((CH,), jnp.int32),
        pltpu.VMEM((CH,), jnp.int32),
        pltpu.VMEM((CH,), jnp.int32),
        pltpu.VMEM((CH,), jnp.int32),
        pltpu.VMEM((CH, OUT), jnp.float32),
        pltpu.VMEM((CH, OUT), jnp.float32),
        pltpu.VMEM((KOC, OUT), jnp.float32),
        pltpu.SemaphoreType.DMA,
        pltpu.SemaphoreType.DMA,
        pltpu.SemaphoreType.DMA,
        pltpu.SemaphoreType.DMA,
        pltpu.VMEM_SHARED((KP, OUT), jnp.float32),
    ],
)
def _sc_messages(xw2_h, gidx2_h, comb2_h, recip_h, aggp_h,
                 recip_v, gidx_v0, gidx_v1, comb_v0, comb_v1, dstr_v0,
                 dstr_v1, rows_v0, rows_v1, stage_v, gsem0, gsem1, ssem0,
                 ssem1, agg_sh):
    c = lax.axis_index("c")
    s = lax.axis_index("s")
    wid = c * NS + s

    def _fill(i, _):
        stage_v[i // (OUT // L),
                pl.ds((i % (OUT // L)) * L, L)] = jnp.zeros((L,), jnp.float32)
        return 0
    lax.fori_loop(0, KOC * (OUT // L), _fill, 0)
    pltpu.sync_copy(stage_v.at[pl.ds(0, KPS)], agg_sh.at[pl.ds(s * KPS, KPS)])
    pltpu.sync_copy(recip_h, recip_v)
    plsc.subcore_barrier()

    base = wid * EPT
    gidx_b = (gidx_v0, gidx_v1)
    comb_b = (comb_v0, comb_v1)
    rows_b = (rows_v0, rows_v1)
    gsem_b = (gsem0, gsem1)
    ssem_b = (ssem0, ssem1)
    dstr_b = (dstr_v0, dstr_v1)

    pltpu.sync_copy(gidx2_h.at[pl.ds(base, CH)], gidx_v0)
    pltpu.sync_copy(comb2_h.at[pl.ds(base, CH)], comb_v0)
    pltpu.async_copy(xw2_h.at[gidx_v0], rows_v0, gsem0)
    for j in range(CH // L):
        dstr_v0[pl.ds(j * L, L)] = jnp.full((L,), K, jnp.int32)
        dstr_v1[pl.ds(j * L, L)] = jnp.full((L,), K, jnp.int32)
    pltpu.async_copy(stage_v.at[pl.ds(0, CH)], agg_sh.at[dstr_v0], ssem0,
                     add=True)
    pltpu.async_copy(stage_v.at[pl.ds(0, CH)], agg_sh.at[dstr_v1], ssem1,
                     add=True)

    def _pair(kk, _):
        for b in range(2):
            k = 2 * kk + b
            nxt = (k + 1) * CH
            nxt = jnp.where(nxt >= EPT, 0, nxt)
            pltpu.sync_copy(gidx2_h.at[pl.ds(base + nxt, CH)],
                            gidx_b[1 - b])
            pltpu.sync_copy(comb2_h.at[pl.ds(base + nxt, CH)],
                            comb_b[1 - b])
            pltpu.make_async_copy(xw2_h.at[gidx_b[b]], rows_b[b],
                                  gsem_b[b]).wait()
            pltpu.make_async_copy(rows_b[1 - b], agg_sh.at[dstr_b[1 - b]],
                                  ssem_b[1 - b]).wait()
            pltpu.async_copy(xw2_h.at[gidx_b[1 - b]], rows_b[1 - b],
                             gsem_b[1 - b])
            for j in range(CH // L):
                sl = pl.ds(j * L, L)
                c16 = comb_b[b][sl]
                dstr_b[b][sl] = lax.shift_right_logical(c16, 5)
                sc16 = plsc.load_gather(recip_v, [c16])
                for r in range(L):
                    spl = jnp.full((L,), sc16[r], jnp.float32)
                    i = j * L + r
                    for d in range(OUT // L):
                        dl = pl.ds(d * L, L)
                        rows_b[b][i, dl] = rows_b[b][i, dl] * spl
            pltpu.async_copy(rows_b[b], agg_sh.at[dstr_b[b]], ssem_b[b],
                             add=True)
        return 0
    lax.fori_loop(0, NCHUNK // 2, _pair, 0)
    pltpu.make_async_copy(xw2_h.at[gidx_v0], rows_v0, gsem0).wait()
    pltpu.make_async_copy(rows_v0, agg_sh.at[dstr_v0], ssem0).wait()
    pltpu.make_async_copy(rows_v1, agg_sh.at[dstr_v1], ssem1).wait()

    plsc.subcore_barrier()

    @pl.when(s < K // KOC)
    def _():
        ksl = pl.ds(s * KOC, KOC)
        pltpu.sync_copy(agg_sh.at[ksl], stage_v)
        pltpu.sync_copy(stage_v, aggp_h.at[c, ksl])


def _mm_body(a_ref, b_ref, o_ref):
    o_ref[...] = jax.lax.dot_general(
        a_ref[...], b_ref[...], (((1,), (0,)), ((), ())),
        precision=_HI, preferred_element_type=jnp.float32)


def _tc_xw2(hp, wall2):
    return pl.pallas_call(
        _mm_body,
        grid=(8,),
        in_specs=[pl.BlockSpec((K, H), lambda i: (0, 0)),
                  pl.BlockSpec((H, 512), lambda i: (0, i))],
        out_specs=pl.BlockSpec((K, 512), lambda i: (0, i)),
        out_shape=jax.ShapeDtypeStruct((K, R * OUT), jnp.float32),
    )(hp, wall2)


def _root_body(a_ref, b_ref, bias_ref, o_ref):
    o_ref[...] = jax.lax.dot_general(
        a_ref[...], b_ref[...], (((1,), (0,)), ((), ())),
        precision=_HI, preferred_element_type=jnp.float32) + bias_ref[...]


def _tc_root(hp, root, bias):
    return pl.pallas_call(
        _root_body,
        in_specs=[pl.BlockSpec((K, H), lambda: (0, 0)),
                  pl.BlockSpec((H, OUT), lambda: (0, 0)),
                  pl.BlockSpec((1, OUT), lambda: (0, 0))],
        out_specs=pl.BlockSpec((K, OUT), lambda: (0, 0)),
        out_shape=jax.ShapeDtypeStruct((K, OUT), jnp.float32),
    )(hp, root, bias.reshape(1, OUT))


def _recip_body(c_ref, o_ref):
    cnt = c_ref[0] + c_ref[1]
    i0 = lax.broadcasted_iota(jnp.int32, (NB2P // 128, 128), 0)
    i1 = lax.broadcasted_iota(jnp.int32, (NB2P // 128, 128), 1)
    flat = i0 * 128 + i1
    r = 1.0 / jnp.maximum(cnt, 1.0)
    o_ref[...] = jnp.where(flat < NB2, r, 0.0)


def _tc_recip(cnt2p):
    return pl.pallas_call(
        _recip_body,
        in_specs=[pl.BlockSpec((NC, NB2P // 128, 128), lambda: (0, 0, 0))],
        out_specs=pl.BlockSpec((NB2P // 128, 128), lambda: (0, 0)),
        out_shape=jax.ShapeDtypeStruct((NB2P // 128, 128), jnp.float32),
    )(cnt2p.reshape(NC, NB2P // 128, 128)).reshape(NB2P)


_FC = 4096
_NF = (K * OUT) // _FC


def _fin_body(w_ref, p0_ref, p1_ref, hr_ref, b_ref, o_ref):
    i = pl.program_id(0)
    f = p0_ref[...] + p1_ref[...] + hr_ref[...]
    part = jax.lax.dot_general(
        w_ref[...], f, (((1,), (0,)), ((), ())),
        precision=_HI, preferred_element_type=jnp.float32)

    @pl.when(i == 0)
    def _():
        o_ref[...] = jnp.zeros_like(o_ref)

    o_ref[...] += part

    @pl.when(i == _NF - 1)
    def _():
        o_ref[...] += b_ref[...]


def _tc_final(lin_W, p0, p1, hrootflat, lin_b):
    return pl.pallas_call(
        _fin_body,
        grid=(_NF,),
        in_specs=[pl.BlockSpec((128, _FC), lambda i: (0, i)),
                  pl.BlockSpec((_FC, 1), lambda i: (i, 0)),
                  pl.BlockSpec((_FC, 1), lambda i: (i, 0)),
                  pl.BlockSpec((_FC, 1), lambda i: (i, 0)),
                  pl.BlockSpec((128, 1), lambda i: (0, 0))],
        out_specs=pl.BlockSpec((128, 1), lambda i: (0, 0)),
        out_shape=jax.ShapeDtypeStruct((128, 1), jnp.float32),
    )(lin_W, p0, p1, hrootflat, lin_b.reshape(128, 1))


def kernel(x, edge_index, edge_type, conv1_W, conv1_root, conv1_bias,
           gnn_rel_W, gnn_rel_b, gnn_root_W, conv2_W, conv2_root, conv2_bias,
           lin_W, lin_b):
    src, dst = edge_index[0], edge_index[1]

    valid0 = jnp.ones((E,), dtype=bool)
    xw = jnp.einsum('nc,rcd->nrd', x, conv1_W)
    m = xw[src, edge_type]
    comb = jnp.where(valid0, dst * R + edge_type, 0)
    ones = valid0.astype(x.dtype)
    cnt = jax.ops.segment_sum(ones, comb, num_segments=N * R)
    denom = jnp.maximum(cnt[comb], 1.0)
    m = m * (ones / denom)[:, None]
    agg = jax.ops.segment_sum(m, jnp.where(valid0, dst, 0), num_segments=N)
    h = agg + x @ conv1_root + conv1_bias

    agg2 = jax.ops.segment_sum(h[src], dst, num_segments=N)
    score = (agg2 @ gnn_rel_W + gnn_rel_b + h @ gnn_root_W).reshape(-1)
    score = jnp.tanh(score)
    topv, perm = jax.lax.top_k(score, K)
    hp = h[perm] * topv[:, None]
    new_id = jnp.full((N,), -1, jnp.int32).at[perm].set(
        jnp.arange(K, dtype=jnp.int32))

    nid_pad = jnp.full((NIDP,), -1, jnp.int32).at[:N].set(new_id)
    comb2, gidx2, cnt2p = _sc_relabel(src, dst, edge_type, nid_pad)
    recip2 = _tc_recip(cnt2p.reshape(NC, NB2P))

    wall2 = jnp.transpose(conv2_W, (1, 0, 2)).reshape(H, R * OUT)
    xw2 = _tc_xw2(hp, wall2).reshape(K * R, OUT)
    hproot = _tc_root(hp, conv2_root, conv2_bias)

    aggp = _sc_messages(xw2, gidx2, comb2, recip2)

    p0 = aggp[0].reshape(K * OUT, 1)
    p1 = aggp[1].reshape(K * OUT, 1)
    hrf = hproot.reshape(K * OUT, 1)
    out = _tc_final(lin_W, p0, p1, hrf, lin_b)
    return out.reshape(128)
